# Initial kernel scaffold; baseline (speedup 1.0000x reference)
import jax
import jax.numpy as jnp
from jax.experimental import pallas as pl


def kernel(user_age, user_gender, item_id, item_cate, item_id_seq, item_cate_seq, userid, neighbor_ids, user_age_table, user_gender_table, item_id_table, item_cate_table, user_mem_0_table, user_mem_1_table):
    raise NotImplementedError("write your pallas kernel here")


# speedup vs baseline: 3.5702x; 3.5702x over previous
import jax
import jax.numpy as jnp
from jax import lax
from jax.experimental import pallas as pl
from jax.experimental.pallas import tpu as pltpu
from jax.experimental.pallas import tpu_sc as plsc

NC, NS = 2, 16
NW = NC * NS

B = 4096
L = 200
NN = 50
SEQ = B * L
NEI = B * NN

C32 = 1600
C64 = 800
SB = B // NW


def _body(user_age, user_gender, item_id, item_cate, seq_id_idx, seq_cate_idx,
          userid, neigh_idx,
          age_tab, gender_tab, item_tab, cate_tab, mem0_tab, mem1_tab,
          user_emb, item_emb, seq_emb, mem0_out, mem1_out, neigh0_out, neigh1_out,
          idx_big, idx_mid, idx_small, rows32, rows64, s32, s64, sem):
    wid = lax.axis_index("s") * NC + lax.axis_index("c")

    def gather_chunks(table, idx_hbm, out_hbm, col, width, per_tile, chunk,
                      idx_v, rows_v):
        base0 = wid * per_tile
        for i in range(per_tile // chunk):
            base = base0 + i * chunk
            pltpu.sync_copy(idx_hbm.at[pl.ds(base, chunk)], idx_v)
            pltpu.async_copy(table.at[idx_v], rows_v, sem).wait()
            if col == 0 and width == out_hbm.shape[1]:
                dst = out_hbm.at[pl.ds(base, chunk)]
            else:
                dst = out_hbm.at[pl.ds(base, chunk), pl.ds(col, width)]
            pltpu.sync_copy(rows_v, dst)

    gather_chunks(item_tab, seq_id_idx, seq_emb, 0, 32, SEQ // NW, C32,
                  idx_big, rows32)
    gather_chunks(cate_tab, seq_cate_idx, seq_emb, 32, 32, SEQ // NW, C32,
                  idx_big, rows32)
    gather_chunks(mem0_tab, neigh_idx, neigh0_out, 0, 64, NEI // NW, C64,
                  idx_mid, rows64)
    gather_chunks(mem1_tab, neigh_idx, neigh1_out, 0, 32, NEI // NW, C32,
                  idx_big, rows32)
    gather_chunks(age_tab, user_age, user_emb, 0, 32, SB, SB, idx_small, s32)
    gather_chunks(gender_tab, user_gender, user_emb, 32, 32, SB, SB,
                  idx_small, s32)
    gather_chunks(item_tab, item_id, item_emb, 0, 32, SB, SB, idx_small, s32)
    gather_chunks(cate_tab, item_cate, item_emb, 32, 32, SB, SB,
                  idx_small, s32)
    gather_chunks(mem0_tab, userid, mem0_out, 0, 64, SB, SB, idx_small, s64)
    gather_chunks(mem1_tab, userid, mem1_out, 0, 32, SB, SB, idx_small, s32)


def kernel(user_age, user_gender, item_id, item_cate, item_id_seq,
           item_cate_seq, userid, neighbor_ids, user_age_table,
           user_gender_table, item_id_table, item_cate_table,
           user_mem_0_table, user_mem_1_table):
    seq_id = item_id_seq.reshape(SEQ)
    seq_cate = item_cate_seq.reshape(SEQ)
    neigh = neighbor_ids.reshape(NEI)

    mesh = plsc.VectorSubcoreMesh(core_axis_name="c", subcore_axis_name="s")
    out_type = (
        jax.ShapeDtypeStruct((B, 64), jnp.float32),
        jax.ShapeDtypeStruct((B, 64), jnp.float32),
        jax.ShapeDtypeStruct((SEQ, 64), jnp.float32),
        jax.ShapeDtypeStruct((B, 64), jnp.float32),
        jax.ShapeDtypeStruct((B, 32), jnp.float32),
        jax.ShapeDtypeStruct((NEI, 64), jnp.float32),
        jax.ShapeDtypeStruct((NEI, 32), jnp.float32),
    )
    scratch = [
        pltpu.VMEM((C32,), jnp.int32),
        pltpu.VMEM((C64,), jnp.int32),
        pltpu.VMEM((SB,), jnp.int32),
        pltpu.VMEM((C32, 32), jnp.float32),
        pltpu.VMEM((C64, 64), jnp.float32),
        pltpu.VMEM((SB, 32), jnp.float32),
        pltpu.VMEM((SB, 64), jnp.float32),
        pltpu.SemaphoreType.DMA,
    ]
    f = pl.kernel(_body, out_type=out_type, mesh=mesh, scratch_types=scratch,
                  compiler_params=pltpu.CompilerParams(
                      use_tc_tiling_on_sc=False))
    user_emb, item_emb, seq_emb, m0, m1, n0, n1 = f(
        user_age, user_gender, item_id, item_cate, seq_id, seq_cate, userid,
        neigh, user_age_table, user_gender_table, item_id_table,
        item_cate_table, user_mem_0_table, user_mem_1_table)
    return (user_emb, item_emb, seq_emb.reshape(B, L, 64), m0, m1,
            n0.reshape(B, NN, 64), n1.reshape(B, NN, 32))


# speedup vs baseline: 3.6464x; 1.0214x over previous
import jax
import jax.numpy as jnp
from jax import lax
from jax.experimental import pallas as pl
from jax.experimental.pallas import tpu as pltpu
from jax.experimental.pallas import tpu_sc as plsc

NC, NS = 2, 16
NW = NC * NS

B = 4096
L = 200
NN = 50
SEQ = B * L
NEI = B * NN

C32 = 1280
C64 = 320
SB = B // NW


def _body(user_age, user_gender, item_id, item_cate, seq_id_idx, seq_cate_idx,
          userid, neigh_idx,
          age_tab, gender_tab, item_tab, cate_tab, mem0_tab, mem1_tab,
          user_emb, item_emb, seq_emb, mem0_out, mem1_out, neigh0_out, neigh1_out,
          i32a, i32b, i64a, i64b, isml, r32a, r32b, r64a, r64b,
          gsem, wsem):
    wid = lax.axis_index("s") * NC + lax.axis_index("c")

    def dst_slice(out_hbm, base, chunk, col, width):
        if col == 0 and width == out_hbm.shape[1]:
            return out_hbm.at[pl.ds(base, chunk)]
        return out_hbm.at[pl.ds(base, chunk), pl.ds(col, width)]

    def run_job(table, idx_hbm, out_hbm, col, width, per_tile, chunk,
                idx_bufs, row_bufs):
        n = per_tile // chunk
        base0 = wid * per_tile
        gh = [None] * n
        wh = [None] * n
        pltpu.sync_copy(idx_hbm.at[pl.ds(base0, chunk)], idx_bufs[0])
        gh[0] = pltpu.async_copy(table.at[idx_bufs[0]], row_bufs[0], gsem)
        if n > 1:
            pltpu.sync_copy(idx_hbm.at[pl.ds(base0 + chunk, chunk)],
                            idx_bufs[1])
        for i in range(n):
            gh[i].wait()
            if i >= 1:
                wh[i - 1].wait()
            if i + 1 < n:
                gh[i + 1] = pltpu.async_copy(
                    table.at[idx_bufs[(i + 1) % 2]], row_bufs[(i + 1) % 2],
                    gsem)
            wh[i] = pltpu.async_copy(
                row_bufs[i % 2],
                dst_slice(out_hbm, base0 + i * chunk, chunk, col, width),
                wsem)
            if i + 2 < n:
                pltpu.sync_copy(
                    idx_hbm.at[pl.ds(base0 + (i + 2) * chunk, chunk)],
                    idx_bufs[i % 2])
        wh[n - 1].wait()

    def run_small(table, idx_hbm, out_hbm, col, width, rows_v):
        base = wid * SB
        pltpu.sync_copy(idx_hbm.at[pl.ds(base, SB)], isml)
        pltpu.async_copy(table.at[isml], rows_v, gsem).wait()
        pltpu.sync_copy(rows_v, dst_slice(out_hbm, base, SB, col, width))

    run_job(item_tab, seq_id_idx, seq_emb, 0, 32, SEQ // NW, C32,
            (i32a, i32b), (r32a, r32b))
    run_job(cate_tab, seq_cate_idx, seq_emb, 32, 32, SEQ // NW, C32,
            (i32a, i32b), (r32a, r32b))
    run_job(mem0_tab, neigh_idx, neigh0_out, 0, 64, NEI // NW, C64,
            (i64a, i64b), (r64a, r64b))
    run_job(mem1_tab, neigh_idx, neigh1_out, 0, 32, NEI // NW, C32,
            (i32a, i32b), (r32a, r32b))
    sml32 = r32a.at[pl.ds(0, SB)]
    sml64 = r64a.at[pl.ds(0, SB)]
    run_small(age_tab, user_age, user_emb, 0, 32, sml32)
    run_small(gender_tab, user_gender, user_emb, 32, 32, sml32)
    run_small(item_tab, item_id, item_emb, 0, 32, sml32)
    run_small(cate_tab, item_cate, item_emb, 32, 32, sml32)
    run_small(mem0_tab, userid, mem0_out, 0, 64, sml64)
    run_small(mem1_tab, userid, mem1_out, 0, 32, sml32)


def kernel(user_age, user_gender, item_id, item_cate, item_id_seq,
           item_cate_seq, userid, neighbor_ids, user_age_table,
           user_gender_table, item_id_table, item_cate_table,
           user_mem_0_table, user_mem_1_table):
    seq_id = item_id_seq.reshape(SEQ)
    seq_cate = item_cate_seq.reshape(SEQ)
    neigh = neighbor_ids.reshape(NEI)

    mesh = plsc.VectorSubcoreMesh(core_axis_name="c", subcore_axis_name="s")
    out_type = (
        jax.ShapeDtypeStruct((B, 64), jnp.float32),
        jax.ShapeDtypeStruct((B, 64), jnp.float32),
        jax.ShapeDtypeStruct((SEQ, 64), jnp.float32),
        jax.ShapeDtypeStruct((B, 64), jnp.float32),
        jax.ShapeDtypeStruct((B, 32), jnp.float32),
        jax.ShapeDtypeStruct((NEI, 64), jnp.float32),
        jax.ShapeDtypeStruct((NEI, 32), jnp.float32),
    )
    scratch = [
        pltpu.VMEM((C32,), jnp.int32),
        pltpu.VMEM((C32,), jnp.int32),
        pltpu.VMEM((C64,), jnp.int32),
        pltpu.VMEM((C64,), jnp.int32),
        pltpu.VMEM((SB,), jnp.int32),
        pltpu.VMEM((C32, 32), jnp.float32),
        pltpu.VMEM((C32, 32), jnp.float32),
        pltpu.VMEM((C64, 64), jnp.float32),
        pltpu.VMEM((C64, 64), jnp.float32),
        pltpu.SemaphoreType.DMA,
        pltpu.SemaphoreType.DMA,
    ]
    f = pl.kernel(_body, out_type=out_type, mesh=mesh, scratch_types=scratch,
                  compiler_params=pltpu.CompilerParams(
                      use_tc_tiling_on_sc=False))
    user_emb, item_emb, seq_emb, m0, m1, n0, n1 = f(
        user_age, user_gender, item_id, item_cate, seq_id, seq_cate, userid,
        neigh, user_age_table, user_gender_table, item_id_table,
        item_cate_table, user_mem_0_table, user_mem_1_table)
    return (user_emb, item_emb, seq_emb.reshape(B, L, 64), m0, m1,
            n0.reshape(B, NN, 64), n1.reshape(B, NN, 32))
